# baseline (speedup 1.0000x reference)
import functools

import jax
import jax.numpy as jnp
from jax import lax
from jax.experimental import pallas as pl
from jax.experimental.pallas import tpu as pltpu
from jax.experimental.pallas import tpu_sc as plsc

N = 819200
M = 32
L = 512
LANES = 16

_info = plsc.get_sparse_core_info()
NC = _info.num_cores
NS = _info.num_subcores
NW = NC * NS

ROWS_PER_W = N // NW
CHUNK = 1024
N_CHUNKS = ROWS_PER_W // CHUNK
GSPLIT = 128


def _sc_call(embedding, tok, bbasis, acoefft):
  mesh = plsc.VectorSubcoreMesh(core_axis_name="c", subcore_axis_name="s")

  dnums = lax.GatherDimensionNumbers(
      offset_dims=(), collapsed_slice_dims=(0,), start_index_map=(0,))

  def _shuffle(v, idx):
    return lax.gather(v, idx[:, None], dnums, (1,),
                      mode=lax.GatherScatterMode.PROMISE_IN_BOUNDS)

  @functools.partial(
      pl.kernel,
      mesh=mesh,
      out_type=jax.ShapeDtypeStruct((N * M,), jnp.float32),
      scratch_types=[
          pltpu.VMEM((CHUNK,), jnp.int32),
          pltpu.VMEM((CHUNK, M), jnp.float32),
          pltpu.VMEM((CHUNK * M,), jnp.float32),
          pltpu.VMEM((L, M), jnp.float32),
          pltpu.VMEM((L, M), jnp.float32),
          pltpu.SemaphoreType.DMA,
      ],
      compiler_params=pltpu.CompilerParams(use_tc_tiling_on_sc=False),
  )
  def k(emb_hbm, tok_hbm, b_hbm, a_hbm, out_hbm, idx_v, rows_v, out_v, b_v,
        a_v, sem):
    wid = lax.axis_index("s") * NC + lax.axis_index("c")
    slab = wid * ROWS_PER_W

    pltpu.sync_copy(b_hbm, b_v)
    pltpu.sync_copy(a_hbm, a_v)

    lane = lax.iota(jnp.int32, LANES)
    perms = [jnp.bitwise_xor(lane, k) for k in (8, 4, 2, 1)]

    def row_body(r, _):
      j = jnp.bitwise_and(r, L - 1)
      x0 = rows_v[r, pl.ds(0, LANES)]
      x1 = rows_v[r, pl.ds(LANES, LANES)]
      b0 = b_v[j, pl.ds(0, LANES)]
      b1 = b_v[j, pl.ds(LANES, LANES)]
      t = x0 * b0 + x1 * b1
      for p in perms:
        t = t + _shuffle(t, p)
      a0 = a_v[j, pl.ds(0, LANES)]
      a1 = a_v[j, pl.ds(LANES, LANES)]
      out_v[pl.ds(r * M, LANES)] = t * a0
      out_v[pl.ds(r * M + LANES, LANES)] = t * a1
      return _

    for c in range(N_CHUNKS):
      base = slab + c * CHUNK
      pltpu.sync_copy(tok_hbm.at[pl.ds(base, CHUNK)], idx_v)
      copies = []
      for g in range(CHUNK // GSPLIT):
        copies.append(
            pltpu.async_copy(
                emb_hbm.at[idx_v.at[pl.ds(g * GSPLIT, GSPLIT)]],
                rows_v.at[pl.ds(g * GSPLIT, GSPLIT)],
                sem,
            ))
      for cp in copies:
        cp.wait()
      lax.fori_loop(0, CHUNK, row_body, None)
      pltpu.sync_copy(out_v, out_hbm.at[pl.ds(base * M, CHUNK * M)])

  return k(embedding, tok, bbasis, acoefft)


def kernel(k_tensor, token_indices, embedding, Acoeff, Bbasis):
  del k_tensor
  tok = token_indices.astype(jnp.int32)
  acoefft = Acoeff.T
  return _sc_call(embedding, tok, Bbasis, acoefft).reshape(N, M)


# speedup vs baseline: 1.7845x; 1.7845x over previous
import functools

import jax
import jax.numpy as jnp
from jax import lax
from jax.experimental import pallas as pl
from jax.experimental.pallas import tpu as pltpu
from jax.experimental.pallas import tpu_sc as plsc

N = 819200
M = 32
L = 512
LANES = 16

_info = plsc.get_sparse_core_info()
NC = _info.num_cores
NS = _info.num_subcores
NW = NC * NS

ROWS_PER_W = N // NW
CH = 1024
NCH = ROWS_PER_W // CH

PACK = 128 // M
NP = N // PACK
COMB = 2048
TCBLK = PACK * COMB
NBLK = N // TCBLK


def _sc_gather(embedding, tok):
  mesh = plsc.VectorSubcoreMesh(core_axis_name="c", subcore_axis_name="s")

  @functools.partial(
      pl.kernel,
      mesh=mesh,
      out_type=jax.ShapeDtypeStruct((N, M), jnp.float32),
      scratch_types=[
          pltpu.VMEM((CH,), jnp.int32),
          pltpu.VMEM((CH,), jnp.int32),
          pltpu.VMEM((CH, M), jnp.float32),
          pltpu.VMEM((CH, M), jnp.float32),
          pltpu.VMEM((CH,), jnp.int32),
          pltpu.VMEM((CH,), jnp.int32),
          pltpu.VMEM((CH,), jnp.int32),
          pltpu.SemaphoreType.DMA,
          pltpu.SemaphoreType.DMA,
          pltpu.SemaphoreType.DMA,
          pltpu.SemaphoreType.DMA,
      ],
      compiler_params=pltpu.CompilerParams(use_tc_tiling_on_sc=False),
  )
  def k(emb_hbm, tok_hbm, out_hbm, idx0, idx1, rows0, rows1, di0, di1, patt,
        sg0, sg1, so0, so1):
    wid = lax.axis_index("s") * NC + lax.axis_index("c")
    slab = wid * ROWS_PER_W
    idx = (idx0, idx1)
    rows = (rows0, rows1)
    di = (di0, di1)
    sg = (sg0, sg1)
    so = (so0, so1)

    lane = lax.iota(jnp.int32, LANES)

    def patt_body(r, _):
      patt[pl.ds(r * LANES, LANES)] = (r * LANES + lane) * PACK
      return _

    lax.fori_loop(0, CH // LANES, patt_body, None)

    def fill_di(dref, d0):
      def body(r, _):
        dref[pl.ds(r * LANES, LANES)] = patt[pl.ds(r * LANES, LANES)] + d0
        return _

      lax.fori_loop(0, CH // LANES, body, None)

    def dst_base(c):
      s = slab + c * CH
      return ((s & ~(TCBLK - 1)) + PACK * (s & (COMB - 1))
              + ((s >> 11) & (PACK - 1)))

    gath = {}
    wout = {}
    pltpu.sync_copy(tok_hbm.at[pl.ds(slab, CH)], idx0)
    gath[0] = pltpu.async_copy(emb_hbm.at[idx0], rows0, sg0)
    for c in range(NCH):
      b = c & 1
      if c + 1 < NCH:
        nb = 1 - b
        pltpu.sync_copy(tok_hbm.at[pl.ds(slab + (c + 1) * CH, CH)], idx[nb])
        if c >= 1:
          wout[c - 1].wait()
        gath[c + 1] = pltpu.async_copy(emb_hbm.at[idx[nb]], rows[nb], sg[nb])
      fill_di(di[b], dst_base(c))
      gath[c].wait()
      wout[c] = pltpu.async_copy(rows[b], out_hbm.at[di[b]], so[b])
    wout[NCH - 2].wait()
    wout[NCH - 1].wait()

  return k(embedding, tok)


def _tc_finish(x2, bp4, ap4, kmat):
  def body(x_ref, bp_ref, ap_ref, k_ref, o_ref):
    bpt = jnp.tile(bp_ref[...], (PACK, 1))
    apt = jnp.tile(ap_ref[...], (PACK, 1))
    t = x_ref[...] * bpt
    u = lax.dot(t, k_ref[...], preferred_element_type=jnp.float32)
    ua = u * apt
    for q in range(PACK):
      o_ref[pl.ds(q * COMB, COMB), :] = ua[:, q * M:(q + 1) * M]

  return pl.pallas_call(
      body,
      grid=(NBLK,),
      in_specs=[
          pl.BlockSpec((COMB, 128), lambda i: (i, 0)),
          pl.BlockSpec((L, 128), lambda i: (0, 0)),
          pl.BlockSpec((L, 128), lambda i: (0, 0)),
          pl.BlockSpec((128, 128), lambda i: (0, 0)),
      ],
      out_specs=pl.BlockSpec((TCBLK, M), lambda i: (i, 0)),
      out_shape=jax.ShapeDtypeStruct((N, M), jnp.float32),
  )(x2, bp4, ap4, kmat)


def kernel(k_tensor, token_indices, embedding, Acoeff, Bbasis):
  del k_tensor
  tok = token_indices.astype(jnp.int32)
  xg = _sc_gather(embedding, tok)
  x2 = xg.reshape(NP, 128)
  bp4 = jnp.tile(Bbasis, (1, PACK))
  ap4 = jnp.tile(Acoeff.T, (1, PACK))
  seg = jnp.arange(128, dtype=jnp.int32) // M
  kmat = (seg[:, None] == seg[None, :]).astype(jnp.float32)
  return _tc_finish(x2, bp4, ap4, kmat)
